# Initial kernel scaffold; baseline (speedup 1.0000x reference)
import functools

import jax
import jax.numpy as jnp
import numpy as np
from jax import lax
from jax.experimental import pallas as pl
from jax.experimental.pallas import tpu as pltpu
from jax.experimental.pallas import tpu_sc as plsc

_NUM_IT = 3

_C_ADD_LO = np.uint32(0x7F4A7C15)
_C_ADD_HI = np.uint32(0x9E3779B9)
_M1_LO = np.uint32(0x1CE4E5B9)
_M1_HI = np.uint32(0xBF58476D)
_M2_LO = np.uint32(0x133111EB)
_M2_HI = np.uint32(0x94D049BB)
_C_NBR = np.uint32(0x1B873593)
_OWN_LO_MUL = np.uint32(0x1B3)

_U32 = jnp.uint32
_I32 = jnp.int32


def _mulhi_u32(a, b):
    m16 = np.uint32(0xFFFF)
    a0 = a & m16
    a1 = a >> np.uint32(16)
    b0 = b & m16
    b1 = b >> np.uint32(16)
    t = a0 * b0
    mid1 = a1 * b0
    mid2 = a0 * b1
    cross = (t >> np.uint32(16)) + (mid1 & m16) + (mid2 & m16)
    return a1 * b1 + (mid1 >> np.uint32(16)) + (mid2 >> np.uint32(16)) + (
        cross >> np.uint32(16))


def _add64(alo, ahi, blo, bhi):
    lo = alo + blo
    carry = (lo < alo).astype(_U32)
    return lo, ahi + bhi + carry


def _mul64_const(alo, ahi, clo, chi):
    lo = alo * clo
    hi = _mulhi_u32(alo, clo) + alo * chi + ahi * clo
    return lo, hi


def _xorshr64(lo, hi, k):
    ku = np.uint32(k)
    kc = np.uint32(32 - k)
    nlo = lo ^ ((lo >> ku) | (hi << kc))
    nhi = hi ^ (hi >> ku)
    return nlo, nhi


def _mix64(lo, hi):
    lo, hi = _add64(lo, hi, _C_ADD_LO, _C_ADD_HI)
    lo, hi = _xorshr64(lo, hi, 30)
    lo, hi = _mul64_const(lo, hi, _M1_LO, _M1_HI)
    lo, hi = _xorshr64(lo, hi, 27)
    lo, hi = _mul64_const(lo, hi, _M2_LO, _M2_HI)
    lo, hi = _xorshr64(lo, hi, 31)
    return lo, hi


def _limbs_body(colors_ref, l0_ref, l1_ref, l2_ref, l3_ref):
    c = colors_ref[...].astype(_U32)
    lo, hi = _mix64(c + _C_NBR, jnp.zeros_like(c))
    m11 = np.uint32(0x7FF)
    l0_ref[...] = lo & m11
    l1_ref[...] = (lo >> np.uint32(11)) & m11
    l2_ref[...] = lo >> np.uint32(22)
    l3_ref[...] = hi


def _sig_body(colors_ref, *refs):
    planes = refs[:16]
    siglo_ref, sighi_ref = refs[16], refs[17]
    agg_lo = None
    agg_hi = None
    for c in range(4):
        s0 = planes[4 * c + 0][...]
        s1 = planes[4 * c + 1][...]
        s2 = planes[4 * c + 2][...]
        s3 = planes[4 * c + 3][...]
        a = s1 << np.uint32(11)
        b = s2 << np.uint32(22)
        lo1 = s0 + a
        c1 = (lo1 < s0).astype(_U32)
        lo2 = lo1 + b
        c2 = (lo2 < lo1).astype(_U32)
        hic = (s1 >> np.uint32(21)) + (s2 >> np.uint32(10)) + c1 + c2 + s3
        if agg_lo is None:
            agg_lo, agg_hi = lo2, hic
        else:
            agg_lo, agg_hi = _add64(agg_lo, agg_hi, lo2, hic)
    col = colors_ref[...].astype(_U32)
    own_lo = col * _OWN_LO_MUL
    own_hi = col << np.uint32(8)
    vlo, vhi = _add64(own_lo, own_hi, agg_lo, agg_hi)
    slo, shi = _mix64(vlo, vhi)
    siglo_ref[...] = slo
    sighi_ref[...] = shi


def _tc_limbs(colors2d):
    shp = jax.ShapeDtypeStruct(colors2d.shape, _U32)
    return pl.pallas_call(
        _limbs_body,
        out_shape=(shp, shp, shp, shp),
    )(colors2d)


def _tc_sig(colors2d, planes):
    shp = jax.ShapeDtypeStruct(colors2d.shape, _U32)
    return pl.pallas_call(
        _sig_body,
        out_shape=(shp, shp),
    )(colors2d, *planes)


@functools.lru_cache(maxsize=None)
def _make_sc_scatter(n_pad, rows_per_tile, e_rows):
    blocks = rows_per_tile // 16
    mesh = plsc.VectorSubcoreMesh(core_axis_name="c", subcore_axis_name="s")

    @functools.partial(
        pl.kernel,
        mesh=mesh,
        out_type=jax.ShapeDtypeStruct((2, 2 * n_pad, 4), _I32),
        scratch_types=[
            pltpu.VMEM((16, 128), _I32),
            pltpu.VMEM((16, 128), _I32),
            pltpu.VMEM((2048, 4), _I32),
            pltpu.VMEM_SHARED((2 * n_pad, 4), _I32),
            pltpu.SemaphoreType.DMA,
            pltpu.SemaphoreType.DMA,
        ],
    )
    def sc(table, rowi, coli, zeros, out, rowbuf, colbuf, gbuf, acc,
           sem_g, sem_s):
        cid = lax.axis_index("c")
        sid = lax.axis_index("s")
        w = cid * 16 + sid

        @pl.when(sid == 0)
        def _():
            pltpu.sync_copy(zeros, acc)

        plsc.subcore_barrier()
        base = w * rows_per_tile

        def blk(i, carry):
            r0 = base + i * 16
            pltpu.sync_copy(rowi.at[pl.ds(r0, 16)], rowbuf)
            pltpu.sync_copy(coli.at[pl.ds(r0, 16)], colbuf)
            hs = [
                pltpu.async_copy(table.at[colbuf.at[j]],
                                 gbuf.at[pl.ds(j * 128, 128)], sem_g)
                for j in range(16)
            ]
            for h in hs:
                h.wait()
            ss = [
                pltpu.async_copy(gbuf.at[pl.ds(j * 128, 128)],
                                 acc.at[rowbuf.at[j]], sem_s, add=True)
                for j in range(16)
            ]
            for h in ss:
                h.wait()
            return carry

        lax.fori_loop(0, blocks, blk, 0)
        plsc.subcore_barrier()

        @pl.when(sid == 0)
        def _():
            pltpu.sync_copy(acc, out.at[cid])

    return sc


def _sc_scatter(table_i32, rowi, coli, zeros, n_pad, rows_per_tile):
    fn = _make_sc_scatter(n_pad, rows_per_tile, rowi.shape[0])
    return fn(table_i32, rowi, coli, zeros)


def kernel(x, edge_index):
    if x.ndim > 1:
        x = jnp.argmax(x, axis=-1)
    n = x.shape[0]
    e = edge_index.shape[1]

    n_pad = ((n + 1023) // 1024) * 1024
    nrows = n_pad // 128
    spare = n_pad - n

    nw = 32
    rows_per_tile = ((e + nw * 2048 - 1) // (nw * 2048)) * 16
    e_rows = nw * rows_per_tile
    e_pad = e_rows * 128
    per_tile = rows_per_tile * 128

    colors = x.astype(_I32)
    row = edge_index[0].astype(_I32)
    col = edge_index[1].astype(_I32)

    npad_e = e_pad - e
    spread = (jnp.arange(npad_e, dtype=_I32) % np.int32(max(spare, 1))
              ) + np.int32(n)
    row_f = jnp.concatenate([row, spread])
    col_f = jnp.concatenate([col, spread])
    epos = jnp.arange(e_pad, dtype=_I32)
    cp = ((epos // np.int32(per_tile)) % np.int32(16)) // np.int32(8)
    row_adj = (row_f + cp * np.int32(n_pad)).reshape(e_rows, 128)
    col_r = col_f.reshape(e_rows, 128)
    zeros = jnp.zeros((2 * n_pad, 4), _I32)

    colors_pad = jnp.zeros((n_pad,), _I32)

    for _ in range(_NUM_IT):
        colors2d = colors_pad.at[:n].set(colors).reshape(nrows, 128)
        l0, l1, l2, l3 = _tc_limbs(colors2d)
        table = jnp.stack(
            [l0.reshape(-1), l1.reshape(-1), l2.reshape(-1), l3.reshape(-1)],
            axis=1)
        table_i32 = lax.bitcast_convert_type(table, _I32)
        out_sc = _sc_scatter(table_i32, row_adj, col_r, zeros, n_pad,
                             rows_per_tile)
        acc_u = lax.bitcast_convert_type(out_sc, _U32)
        planes = []
        for core in range(2):
            for half in range(2):
                blk = acc_u[core, half * n_pad:(half + 1) * n_pad, :]
                for comp in range(4):
                    planes.append(blk[:, comp].reshape(nrows, 128))
        siglo, sighi = _tc_sig(colors2d, planes)
        sig = (sighi.reshape(-1)[:n].astype(jnp.uint64) << np.uint64(32)) | \
            siglo.reshape(-1)[:n].astype(jnp.uint64)
        _, inv = jnp.unique(sig, return_inverse=True, size=n,
                            fill_value=jnp.uint64(0))
        colors = inv.reshape(-1).astype(_I32)

    return colors.astype(jnp.int64)


# speedup vs baseline: 81.3706x; 81.3706x over previous
import functools

import jax
import jax.numpy as jnp
import numpy as np
from jax import lax
from jax.experimental import pallas as pl
from jax.experimental.pallas import tpu as pltpu
from jax.experimental.pallas import tpu_sc as plsc

_NUM_IT = 3

_C_ADD_LO = np.uint32(0x7F4A7C15)
_C_ADD_HI = np.uint32(0x9E3779B9)
_M1_LO = np.uint32(0x1CE4E5B9)
_M1_HI = np.uint32(0xBF58476D)
_M2_LO = np.uint32(0x133111EB)
_M2_HI = np.uint32(0x94D049BB)
_C_NBR = np.uint32(0x1B873593)
_OWN_LO_MUL = np.uint32(0x1B3)

_U32 = jnp.uint32
_I32 = jnp.int32


def _mulhi_u32(a, b):
    m16 = np.uint32(0xFFFF)
    a0 = a & m16
    a1 = a >> np.uint32(16)
    b0 = b & m16
    b1 = b >> np.uint32(16)
    t = a0 * b0
    mid1 = a1 * b0
    mid2 = a0 * b1
    cross = (t >> np.uint32(16)) + (mid1 & m16) + (mid2 & m16)
    return a1 * b1 + (mid1 >> np.uint32(16)) + (mid2 >> np.uint32(16)) + (
        cross >> np.uint32(16))


def _add64(alo, ahi, blo, bhi):
    lo = alo + blo
    carry = (lo < alo).astype(_U32)
    return lo, ahi + bhi + carry


def _mul64_const(alo, ahi, clo, chi):
    lo = alo * clo
    hi = _mulhi_u32(alo, clo) + alo * chi + ahi * clo
    return lo, hi


def _xorshr64(lo, hi, k):
    ku = np.uint32(k)
    kc = np.uint32(32 - k)
    nlo = lo ^ ((lo >> ku) | (hi << kc))
    nhi = hi ^ (hi >> ku)
    return nlo, nhi


def _mix64(lo, hi):
    lo, hi = _add64(lo, hi, _C_ADD_LO, _C_ADD_HI)
    lo, hi = _xorshr64(lo, hi, 30)
    lo, hi = _mul64_const(lo, hi, _M1_LO, _M1_HI)
    lo, hi = _xorshr64(lo, hi, 27)
    lo, hi = _mul64_const(lo, hi, _M2_LO, _M2_HI)
    lo, hi = _xorshr64(lo, hi, 31)
    return lo, hi


def _limbs_body(colors_ref, l0_ref, l1_ref, l2_ref, l3_ref):
    c = colors_ref[...].astype(_U32)
    lo, hi = _mix64(c + _C_NBR, jnp.zeros_like(c))
    m11 = np.uint32(0x7FF)
    l0_ref[...] = lo & m11
    l1_ref[...] = (lo >> np.uint32(11)) & m11
    l2_ref[...] = lo >> np.uint32(22)
    l3_ref[...] = hi


def _sig_body(colors_ref, *refs):
    planes = refs[:16]
    siglo_ref, sighi_ref = refs[16], refs[17]
    agg_lo = None
    agg_hi = None
    for c in range(4):
        s0 = planes[4 * c + 0][...]
        s1 = planes[4 * c + 1][...]
        s2 = planes[4 * c + 2][...]
        s3 = planes[4 * c + 3][...]
        a = s1 << np.uint32(11)
        b = s2 << np.uint32(22)
        lo1 = s0 + a
        c1 = (lo1 < s0).astype(_U32)
        lo2 = lo1 + b
        c2 = (lo2 < lo1).astype(_U32)
        hic = (s1 >> np.uint32(21)) + (s2 >> np.uint32(10)) + c1 + c2 + s3
        if agg_lo is None:
            agg_lo, agg_hi = lo2, hic
        else:
            agg_lo, agg_hi = _add64(agg_lo, agg_hi, lo2, hic)
    col = colors_ref[...].astype(_U32)
    own_lo = col * _OWN_LO_MUL
    own_hi = col << np.uint32(8)
    vlo, vhi = _add64(own_lo, own_hi, agg_lo, agg_hi)
    slo, shi = _mix64(vlo, vhi)
    siglo_ref[...] = slo
    sighi_ref[...] = shi


def _tc_limbs(colors2d):
    shp = jax.ShapeDtypeStruct(colors2d.shape, _U32)
    return pl.pallas_call(
        _limbs_body,
        out_shape=(shp, shp, shp, shp),
    )(colors2d)


def _tc_sig(colors2d, planes):
    shp = jax.ShapeDtypeStruct(colors2d.shape, _U32)
    return pl.pallas_call(
        _sig_body,
        out_shape=(shp, shp),
    )(colors2d, *planes)


@functools.lru_cache(maxsize=None)
def _make_sc_scatter(n_pad, rows_per_tile, e_rows):
    blocks = rows_per_tile // 16
    mesh = plsc.VectorSubcoreMesh(core_axis_name="c", subcore_axis_name="s")
    oshape = jax.ShapeDtypeStruct((4 * n_pad,), _I32)

    @functools.partial(
        pl.kernel,
        mesh=mesh,
        out_type=(oshape, oshape, oshape, oshape),
        scratch_types=[
            pltpu.VMEM((16, 128), _I32),
            pltpu.VMEM((16, 128), _I32),
            [pltpu.VMEM((2048,), _I32)] * 4,
            [pltpu.VMEM_SHARED((2 * n_pad,), _I32)] * 4,
            [pltpu.VMEM_SHARED((n_pad,), _I32)] * 4,
            pltpu.SemaphoreType.DMA,
            pltpu.SemaphoreType.DMA,
        ],
    )
    def sc(t0, t1, t2, t3, rowi, coli, zeros, o0, o1, o2, o3,
           rowbuf, colbuf, gbufs, accs, tss, sem_g, sem_s):
        cid = lax.axis_index("c")
        sid = lax.axis_index("s")
        w = cid * np.int32(16) + sid
        tplanes = [t0, t1, t2, t3]
        outs = [o0, o1, o2, o3]

        zrows = (2 * n_pad) // 16
        z0 = pl.multiple_of(sid * np.int32(zrows), 8)
        trows = n_pad // 16
        tr0 = pl.multiple_of(sid * np.int32(trows), 8)
        for p in range(4):
            pltpu.sync_copy(zeros.at[pl.ds(z0, zrows)],
                            accs[p].at[pl.ds(z0, zrows)])
            pltpu.sync_copy(tplanes[p].at[pl.ds(tr0, trows)],
                            tss[p].at[pl.ds(tr0, trows)])

        plsc.subcore_barrier()
        base = w * np.int32(rows_per_tile)

        def blk(i, r0):
            del i
            r0 = pl.multiple_of(r0, 16)
            pltpu.sync_copy(rowi.at[pl.ds(r0, 16)], rowbuf)
            pltpu.sync_copy(coli.at[pl.ds(r0, 16)], colbuf)

            def jblk(_, jj):
                g0 = pl.multiple_of(jj * np.int32(128), 8)
                hs = [
                    pltpu.async_copy(tss[p].at[colbuf.at[jj]],
                                     gbufs[p].at[pl.ds(g0, 128)], sem_g)
                    for p in range(4)
                ]
                for h in hs:
                    h.wait()
                ss = [
                    pltpu.async_copy(gbufs[p].at[pl.ds(g0, 128)],
                                     accs[p].at[rowbuf.at[jj]], sem_s,
                                     add=True)
                    for p in range(4)
                ]
                for h in ss:
                    h.wait()
                return jj + np.int32(1)

            lax.fori_loop(0, 16, jblk, np.int32(0))
            return r0 + np.int32(16)

        lax.fori_loop(0, blocks, blk, base)
        plsc.subcore_barrier()
        obase = pl.multiple_of(cid * np.int32(2 * n_pad) + z0, 8)
        for p in range(4):
            pltpu.sync_copy(accs[p].at[pl.ds(z0, zrows)],
                            outs[p].at[pl.ds(obase, zrows)])

    return sc


def _sc_scatter(tplanes, rowi, coli, zeros, n_pad, rows_per_tile):
    fn = _make_sc_scatter(n_pad, rows_per_tile, rowi.shape[0])
    return fn(*tplanes, rowi, coli, zeros)


def kernel(x, edge_index):
    if x.ndim > 1:
        x = jnp.argmax(x, axis=-1)
    n = x.shape[0]
    e = edge_index.shape[1]

    n_pad = ((n + 1023) // 1024) * 1024
    nrows = n_pad // 128
    spare = n_pad - n

    nw = 32
    rows_per_tile = ((e + nw * 2048 - 1) // (nw * 2048)) * 16
    e_rows = nw * rows_per_tile
    e_pad = e_rows * 128
    per_tile = rows_per_tile * 128

    colors = x.astype(_I32)
    row = edge_index[0].astype(_I32)
    col = edge_index[1].astype(_I32)

    npad_e = e_pad - e
    spread = (jnp.arange(npad_e, dtype=_I32) % np.int32(max(spare, 1))
              ) + np.int32(n)
    row_f = jnp.concatenate([row, spread])
    col_f = jnp.concatenate([col, spread])
    epos = jnp.arange(e_pad, dtype=_I32)
    cp = ((epos // np.int32(per_tile)) % np.int32(16)) // np.int32(8)
    row_adj = (row_f + cp * np.int32(n_pad)).reshape(e_rows, 128)
    col_r = col_f.reshape(e_rows, 128)
    zeros = jnp.zeros((2 * n_pad,), _I32)

    colors_pad = jnp.zeros((n_pad,), _I32)

    for _ in range(_NUM_IT):
        colors2d = colors_pad.at[:n].set(colors).reshape(nrows, 128)
        limbs = _tc_limbs(colors2d)
        tplanes = [
            lax.bitcast_convert_type(p, _I32).reshape(-1) for p in limbs
        ]
        out_sc = _sc_scatter(tplanes, row_adj, col_r, zeros, n_pad,
                             rows_per_tile)
        outs_u = [lax.bitcast_convert_type(o, _U32) for o in out_sc]
        planes = []
        for core in range(2):
            for half in range(2):
                off = core * 2 * n_pad + half * n_pad
                for comp in range(4):
                    planes.append(
                        outs_u[comp][off:off + n_pad].reshape(nrows, 128))
        siglo, sighi = _tc_sig(colors2d, planes)
        sig = (sighi.reshape(-1)[:n].astype(jnp.uint64) << np.uint64(32)) | \
            siglo.reshape(-1)[:n].astype(jnp.uint64)
        _, inv = jnp.unique(sig, return_inverse=True, size=n,
                            fill_value=jnp.uint64(0))
        colors = inv.reshape(-1).astype(_I32)

    return colors.astype(jnp.int64)


# speedup vs baseline: 530.9358x; 6.5249x over previous
import functools

import jax
import jax.numpy as jnp
import numpy as np
from jax import lax
from jax.experimental import pallas as pl
from jax.experimental.pallas import tpu as pltpu
from jax.experimental.pallas import tpu_sc as plsc

_NUM_IT = 3

_C_ADD_LO = np.uint32(0x7F4A7C15)
_C_ADD_HI = np.uint32(0x9E3779B9)
_M1_LO = np.uint32(0x1CE4E5B9)
_M1_HI = np.uint32(0xBF58476D)
_M2_LO = np.uint32(0x133111EB)
_M2_HI = np.uint32(0x94D049BB)
_C_NBR = np.uint32(0x1B873593)
_OWN_LO_MUL = np.uint32(0x1B3)

_U32 = jnp.uint32
_I32 = jnp.int32


def _mulhi_u32(a, b):
    m16 = np.uint32(0xFFFF)
    a0 = a & m16
    a1 = a >> np.uint32(16)
    b0 = b & m16
    b1 = b >> np.uint32(16)
    t = a0 * b0
    mid1 = a1 * b0
    mid2 = a0 * b1
    cross = (t >> np.uint32(16)) + (mid1 & m16) + (mid2 & m16)
    return a1 * b1 + (mid1 >> np.uint32(16)) + (mid2 >> np.uint32(16)) + (
        cross >> np.uint32(16))


def _add64(alo, ahi, blo, bhi):
    lo = alo + blo
    carry = (lo < alo).astype(_U32)
    return lo, ahi + bhi + carry


def _mul64_const(alo, ahi, clo, chi):
    lo = alo * clo
    hi = _mulhi_u32(alo, clo) + alo * chi + ahi * clo
    return lo, hi


def _xorshr64(lo, hi, k):
    ku = np.uint32(k)
    kc = np.uint32(32 - k)
    nlo = lo ^ ((lo >> ku) | (hi << kc))
    nhi = hi ^ (hi >> ku)
    return nlo, nhi


def _mix64(lo, hi):
    lo, hi = _add64(lo, hi, _C_ADD_LO, _C_ADD_HI)
    lo, hi = _xorshr64(lo, hi, 30)
    lo, hi = _mul64_const(lo, hi, _M1_LO, _M1_HI)
    lo, hi = _xorshr64(lo, hi, 27)
    lo, hi = _mul64_const(lo, hi, _M2_LO, _M2_HI)
    lo, hi = _xorshr64(lo, hi, 31)
    return lo, hi


def _limbs_body(colors_ref, l0_ref, l1_ref, l2_ref, l3_ref):
    c = colors_ref[...].astype(_U32)
    lo, hi = _mix64(c + _C_NBR, jnp.zeros_like(c))
    m11 = np.uint32(0x7FF)
    l0_ref[...] = lo & m11
    l1_ref[...] = (lo >> np.uint32(11)) & m11
    l2_ref[...] = lo >> np.uint32(22)
    l3_ref[...] = hi


def _sig_body(colors_ref, *refs):
    planes = refs[:16]
    siglo_ref, sighi_ref = refs[16], refs[17]
    agg_lo = None
    agg_hi = None
    for c in range(4):
        s0 = planes[4 * c + 0][...]
        s1 = planes[4 * c + 1][...]
        s2 = planes[4 * c + 2][...]
        s3 = planes[4 * c + 3][...]
        a = s1 << np.uint32(11)
        b = s2 << np.uint32(22)
        lo1 = s0 + a
        c1 = (lo1 < s0).astype(_U32)
        lo2 = lo1 + b
        c2 = (lo2 < lo1).astype(_U32)
        hic = (s1 >> np.uint32(21)) + (s2 >> np.uint32(10)) + c1 + c2 + s3
        if agg_lo is None:
            agg_lo, agg_hi = lo2, hic
        else:
            agg_lo, agg_hi = _add64(agg_lo, agg_hi, lo2, hic)
    col = colors_ref[...].astype(_U32)
    own_lo = col * _OWN_LO_MUL
    own_hi = col << np.uint32(8)
    vlo, vhi = _add64(own_lo, own_hi, agg_lo, agg_hi)
    slo, shi = _mix64(vlo, vhi)
    siglo_ref[...] = slo
    sighi_ref[...] = shi


def _tc_limbs(colors2d):
    shp = jax.ShapeDtypeStruct(colors2d.shape, _U32)
    return pl.pallas_call(
        _limbs_body,
        out_shape=(shp, shp, shp, shp),
    )(colors2d)


def _tc_sig(colors2d, planes):
    shp = jax.ShapeDtypeStruct(colors2d.shape, _U32)
    return pl.pallas_call(
        _sig_body,
        out_shape=(shp, shp),
    )(colors2d, *planes)


@functools.lru_cache(maxsize=None)
def _make_sc_scatter(n_pad, rows_per_tile, e_rows):
    blocks = rows_per_tile // 16
    mesh = plsc.VectorSubcoreMesh(core_axis_name="c", subcore_axis_name="s")
    oshape = jax.ShapeDtypeStruct((4 * n_pad,), _I32)

    @functools.partial(
        pl.kernel,
        mesh=mesh,
        out_type=(oshape, oshape, oshape, oshape),
        scratch_types=[
            pltpu.VMEM((16, 128), _I32),
            pltpu.VMEM((16, 128), _I32),
            [pltpu.VMEM((2048,), _I32)] * 4,
            [pltpu.VMEM_SHARED((2 * n_pad,), _I32)] * 4,
            [pltpu.VMEM_SHARED((n_pad,), _I32)] * 4,
            pltpu.SemaphoreType.DMA,
            pltpu.SemaphoreType.DMA,
        ],
    )
    def sc(t0, t1, t2, t3, rowi, coli, zeros, o0, o1, o2, o3,
           rowbuf, colbuf, gbufs, accs, tss, sem_g, sem_s):
        cid = lax.axis_index("c")
        sid = lax.axis_index("s")
        w = cid * np.int32(16) + sid
        tplanes = [t0, t1, t2, t3]
        outs = [o0, o1, o2, o3]

        zrows = (2 * n_pad) // 16
        z0 = pl.multiple_of(sid * np.int32(zrows), 8)
        trows = n_pad // 16
        tr0 = pl.multiple_of(sid * np.int32(trows), 8)
        for p in range(4):
            pltpu.sync_copy(zeros.at[pl.ds(z0, zrows)],
                            accs[p].at[pl.ds(z0, zrows)])
            pltpu.sync_copy(tplanes[p].at[pl.ds(tr0, trows)],
                            tss[p].at[pl.ds(tr0, trows)])

        plsc.subcore_barrier()
        base = w * np.int32(rows_per_tile)

        def blk(i, r0):
            del i
            r0 = pl.multiple_of(r0, 16)
            pltpu.sync_copy(rowi.at[pl.ds(r0, 16)], rowbuf)
            pltpu.sync_copy(coli.at[pl.ds(r0, 16)], colbuf)

            def jblk(_, jj):
                g0 = pl.multiple_of(jj * np.int32(128), 8)
                hs = [
                    pltpu.async_copy(tss[p].at[colbuf.at[jj]],
                                     gbufs[p].at[pl.ds(g0, 128)], sem_g)
                    for p in range(4)
                ]
                for h in hs:
                    h.wait()
                ss = [
                    pltpu.async_copy(gbufs[p].at[pl.ds(g0, 128)],
                                     accs[p].at[rowbuf.at[jj]], sem_s,
                                     add=True)
                    for p in range(4)
                ]
                for h in ss:
                    h.wait()
                return jj + np.int32(1)

            lax.fori_loop(0, 16, jblk, np.int32(0))
            return r0 + np.int32(16)

        lax.fori_loop(0, blocks, blk, base)
        plsc.subcore_barrier()
        obase = pl.multiple_of(cid * np.int32(2 * n_pad) + z0, 8)
        for p in range(4):
            pltpu.sync_copy(accs[p].at[pl.ds(z0, zrows)],
                            outs[p].at[pl.ds(obase, zrows)])

    return sc


def _sc_scatter(tplanes, rowi, coli, zeros, n_pad, rows_per_tile):
    fn = _make_sc_scatter(n_pad, rows_per_tile, rowi.shape[0])
    return fn(*tplanes, rowi, coli, zeros)


def kernel(x, edge_index):
    if x.ndim > 1:
        x = jnp.argmax(x, axis=-1)
    n = x.shape[0]
    e = edge_index.shape[1]

    n_pad = ((n + 1023) // 1024) * 1024
    nrows = n_pad // 128
    spare = n_pad - n

    nw = 32
    rows_per_tile = ((e + nw * 2048 - 1) // (nw * 2048)) * 16
    e_rows = nw * rows_per_tile
    e_pad = e_rows * 128
    per_tile = rows_per_tile * 128

    colors = x.astype(_I32)
    row = edge_index[0].astype(_I32)
    col = edge_index[1].astype(_I32)

    npad_e = e_pad - e
    spread = (jnp.arange(npad_e, dtype=_I32) % np.int32(max(spare, 1))
              ) + np.int32(n)
    row_f = jnp.concatenate([row, spread])
    col_f = jnp.concatenate([col, spread])
    epos = jnp.arange(e_pad, dtype=_I32)
    cp = ((epos // np.int32(per_tile)) % np.int32(16)) // np.int32(8)
    row_adj = (row_f + cp * np.int32(n_pad)).reshape(e_rows, 128)
    col_r = col_f.reshape(e_rows, 128)
    zeros = jnp.zeros((2 * n_pad,), _I32)

    colors_pad = jnp.zeros((n_pad,), _I32)

    for _ in range(_NUM_IT):
        colors2d = colors_pad.at[:n].set(colors).reshape(nrows, 128)
        limbs = _tc_limbs(colors2d)
        tplanes = [
            lax.bitcast_convert_type(p, _I32).reshape(-1) for p in limbs
        ]
        out_sc = _sc_scatter(tplanes, row_adj, col_r, zeros, n_pad,
                             rows_per_tile)
        outs_u = [lax.bitcast_convert_type(o, _U32) for o in out_sc]
        planes = []
        for core in range(2):
            for half in range(2):
                off = core * 2 * n_pad + half * n_pad
                for comp in range(4):
                    planes.append(
                        outs_u[comp][off:off + n_pad].reshape(nrows, 128))
        siglo, sighi = _tc_sig(colors2d, planes)
        hi = sighi.reshape(-1)[:n]
        lo = siglo.reshape(-1)[:n]
        idx = jnp.arange(n, dtype=_I32)
        hi_s, lo_s, idx_s = lax.sort((hi, lo, idx), num_keys=2)
        neq = (hi_s[1:] != hi_s[:-1]) | (lo_s[1:] != lo_s[:-1])
        flags = jnp.concatenate(
            [jnp.zeros((1,), _I32), neq.astype(_I32)])
        ranks = jnp.cumsum(flags, dtype=_I32)
        colors = jnp.zeros((n,), _I32).at[idx_s].set(ranks)

    return colors.astype(jnp.int64)


# speedup vs baseline: 614.1375x; 1.1567x over previous
import functools

import jax
import jax.numpy as jnp
import numpy as np
from jax import lax
from jax.experimental import pallas as pl
from jax.experimental.pallas import tpu as pltpu
from jax.experimental.pallas import tpu_sc as plsc

_NUM_IT = 3

_C_ADD_LO = np.uint32(0x7F4A7C15)
_C_ADD_HI = np.uint32(0x9E3779B9)
_M1_LO = np.uint32(0x1CE4E5B9)
_M1_HI = np.uint32(0xBF58476D)
_M2_LO = np.uint32(0x133111EB)
_M2_HI = np.uint32(0x94D049BB)
_C_NBR = np.uint32(0x1B873593)
_OWN_LO_MUL = np.uint32(0x1B3)

_U32 = jnp.uint32
_I32 = jnp.int32


def _mulhi_u32(a, b):
    m16 = np.uint32(0xFFFF)
    a0 = a & m16
    a1 = a >> np.uint32(16)
    b0 = b & m16
    b1 = b >> np.uint32(16)
    t = a0 * b0
    mid1 = a1 * b0
    mid2 = a0 * b1
    cross = (t >> np.uint32(16)) + (mid1 & m16) + (mid2 & m16)
    return a1 * b1 + (mid1 >> np.uint32(16)) + (mid2 >> np.uint32(16)) + (
        cross >> np.uint32(16))


def _add64(alo, ahi, blo, bhi):
    lo = alo + blo
    carry = (lo < alo).astype(_U32)
    return lo, ahi + bhi + carry


def _mul64_const(alo, ahi, clo, chi):
    lo = alo * clo
    hi = _mulhi_u32(alo, clo) + alo * chi + ahi * clo
    return lo, hi


def _xorshr64(lo, hi, k):
    ku = np.uint32(k)
    kc = np.uint32(32 - k)
    nlo = lo ^ ((lo >> ku) | (hi << kc))
    nhi = hi ^ (hi >> ku)
    return nlo, nhi


def _mix64(lo, hi):
    lo, hi = _add64(lo, hi, _C_ADD_LO, _C_ADD_HI)
    lo, hi = _xorshr64(lo, hi, 30)
    lo, hi = _mul64_const(lo, hi, _M1_LO, _M1_HI)
    lo, hi = _xorshr64(lo, hi, 27)
    lo, hi = _mul64_const(lo, hi, _M2_LO, _M2_HI)
    lo, hi = _xorshr64(lo, hi, 31)
    return lo, hi


def _limbs_body(colors_ref, l0_ref, l1_ref, l2_ref, l3_ref):
    c = colors_ref[...].astype(_U32)
    lo, hi = _mix64(c + _C_NBR, jnp.zeros_like(c))
    m11 = np.uint32(0x7FF)
    l0_ref[...] = lo & m11
    l1_ref[...] = (lo >> np.uint32(11)) & m11
    l2_ref[...] = lo >> np.uint32(22)
    l3_ref[...] = hi


def _sig_body(colors_ref, *refs):
    planes = refs[:16]
    siglo_ref, sighi_ref = refs[16], refs[17]
    agg_lo = None
    agg_hi = None
    for c in range(4):
        s0 = planes[4 * c + 0][...]
        s1 = planes[4 * c + 1][...]
        s2 = planes[4 * c + 2][...]
        s3 = planes[4 * c + 3][...]
        a = s1 << np.uint32(11)
        b = s2 << np.uint32(22)
        lo1 = s0 + a
        c1 = (lo1 < s0).astype(_U32)
        lo2 = lo1 + b
        c2 = (lo2 < lo1).astype(_U32)
        hic = (s1 >> np.uint32(21)) + (s2 >> np.uint32(10)) + c1 + c2 + s3
        if agg_lo is None:
            agg_lo, agg_hi = lo2, hic
        else:
            agg_lo, agg_hi = _add64(agg_lo, agg_hi, lo2, hic)
    col = colors_ref[...].astype(_U32)
    own_lo = col * _OWN_LO_MUL
    own_hi = col << np.uint32(8)
    vlo, vhi = _add64(own_lo, own_hi, agg_lo, agg_hi)
    slo, shi = _mix64(vlo, vhi)
    siglo_ref[...] = slo
    sighi_ref[...] = shi


def _tc_limbs(colors2d):
    shp = jax.ShapeDtypeStruct(colors2d.shape, _U32)
    return pl.pallas_call(
        _limbs_body,
        out_shape=(shp, shp, shp, shp),
    )(colors2d)


def _tc_sig(colors2d, planes):
    shp = jax.ShapeDtypeStruct(colors2d.shape, _U32)
    return pl.pallas_call(
        _sig_body,
        out_shape=(shp, shp),
    )(colors2d, *planes)


@functools.lru_cache(maxsize=None)
def _make_sc_scatter(n_pad, rows_per_tile, e_rows):
    blocks = rows_per_tile // 16
    mesh = plsc.VectorSubcoreMesh(core_axis_name="c", subcore_axis_name="s")
    oshape = jax.ShapeDtypeStruct((4 * n_pad,), _I32)

    @functools.partial(
        pl.kernel,
        mesh=mesh,
        out_type=(oshape, oshape, oshape, oshape),
        scratch_types=[
            pltpu.VMEM((16, 128), _I32),
            pltpu.VMEM((16, 128), _I32),
            [pltpu.VMEM((2048,), _I32)] * 4,
            [pltpu.VMEM_SHARED((2 * n_pad,), _I32)] * 4,
            [pltpu.VMEM_SHARED((n_pad,), _I32)] * 4,
            pltpu.SemaphoreType.DMA,
            pltpu.SemaphoreType.DMA,
        ],
    )
    def sc(t0, t1, t2, t3, rowi, coli, zeros, o0, o1, o2, o3,
           rowbuf, colbuf, gbufs, accs, tss, sem_g, sem_s):
        cid = lax.axis_index("c")
        sid = lax.axis_index("s")
        w = cid * np.int32(16) + sid
        tplanes = [t0, t1, t2, t3]
        outs = [o0, o1, o2, o3]

        zrows = (2 * n_pad) // 16
        z0 = pl.multiple_of(sid * np.int32(zrows), 8)
        trows = n_pad // 16
        tr0 = pl.multiple_of(sid * np.int32(trows), 8)
        for p in range(4):
            pltpu.sync_copy(zeros.at[pl.ds(z0, zrows)],
                            accs[p].at[pl.ds(z0, zrows)])
            pltpu.sync_copy(tplanes[p].at[pl.ds(tr0, trows)],
                            tss[p].at[pl.ds(tr0, trows)])

        plsc.subcore_barrier()
        base = w * np.int32(rows_per_tile)

        def fire_gathers(jj):
            g0 = pl.multiple_of(jj * np.int32(128), 8)
            for p in range(4):
                pltpu.async_copy(tss[p].at[colbuf.at[jj]],
                                 gbufs[p].at[pl.ds(g0, 128)], sem_g)

        def blk(i, r0):
            del i
            r0 = pl.multiple_of(r0, 16)
            pltpu.sync_copy(rowi.at[pl.ds(r0, 16)], rowbuf)
            pltpu.sync_copy(coli.at[pl.ds(r0, 16)], colbuf)
            fire_gathers(np.int32(0))

            def jblk(_, jj):
                g0 = pl.multiple_of(jj * np.int32(128), 8)

                @pl.when(jj < np.int32(15))
                def _():
                    fire_gathers(jj + np.int32(1))

                for p in range(4):
                    pltpu.make_async_copy(
                        tss[p].at[colbuf.at[jj]],
                        gbufs[p].at[pl.ds(g0, 128)], sem_g).wait()
                for p in range(4):
                    pltpu.async_copy(gbufs[p].at[pl.ds(g0, 128)],
                                     accs[p].at[rowbuf.at[jj]], sem_s,
                                     add=True)
                return jj + np.int32(1)

            lax.fori_loop(0, 16, jblk, np.int32(0))
            for p in range(4):
                pltpu.make_async_copy(zeros.at[pl.ds(0, 2048)], gbufs[p],
                                      sem_s).wait()
            return r0 + np.int32(16)

        lax.fori_loop(0, blocks, blk, base)
        plsc.subcore_barrier()
        obase = pl.multiple_of(cid * np.int32(2 * n_pad) + z0, 8)
        for p in range(4):
            pltpu.sync_copy(accs[p].at[pl.ds(z0, zrows)],
                            outs[p].at[pl.ds(obase, zrows)])

    return sc


def _sc_scatter(tplanes, rowi, coli, zeros, n_pad, rows_per_tile):
    fn = _make_sc_scatter(n_pad, rows_per_tile, rowi.shape[0])
    return fn(*tplanes, rowi, coli, zeros)


def kernel(x, edge_index):
    if x.ndim > 1:
        x = jnp.argmax(x, axis=-1)
    n = x.shape[0]
    e = edge_index.shape[1]

    n_pad = ((n + 1023) // 1024) * 1024
    nrows = n_pad // 128
    spare = n_pad - n

    nw = 32
    rows_per_tile = ((e + nw * 2048 - 1) // (nw * 2048)) * 16
    e_rows = nw * rows_per_tile
    e_pad = e_rows * 128
    per_tile = rows_per_tile * 128

    colors = x.astype(_I32)
    row = edge_index[0].astype(_I32)
    col = edge_index[1].astype(_I32)

    npad_e = e_pad - e
    spread = (jnp.arange(npad_e, dtype=_I32) % np.int32(max(spare, 1))
              ) + np.int32(n)
    row_f = jnp.concatenate([row, spread])
    col_f = jnp.concatenate([col, spread])
    epos = jnp.arange(e_pad, dtype=_I32)
    cp = ((epos // np.int32(per_tile)) % np.int32(16)) // np.int32(8)
    row_adj = (row_f + cp * np.int32(n_pad)).reshape(e_rows, 128)
    col_r = col_f.reshape(e_rows, 128)
    zeros = jnp.zeros((2 * n_pad,), _I32)

    colors_pad = jnp.zeros((n_pad,), _I32)

    for _ in range(_NUM_IT):
        colors2d = colors_pad.at[:n].set(colors).reshape(nrows, 128)
        limbs = _tc_limbs(colors2d)
        tplanes = [
            lax.bitcast_convert_type(p, _I32).reshape(-1) for p in limbs
        ]
        out_sc = _sc_scatter(tplanes, row_adj, col_r, zeros, n_pad,
                             rows_per_tile)
        outs_u = [lax.bitcast_convert_type(o, _U32) for o in out_sc]
        planes = []
        for core in range(2):
            for half in range(2):
                off = core * 2 * n_pad + half * n_pad
                for comp in range(4):
                    planes.append(
                        outs_u[comp][off:off + n_pad].reshape(nrows, 128))
        siglo, sighi = _tc_sig(colors2d, planes)
        hi = sighi.reshape(-1)[:n]
        lo = siglo.reshape(-1)[:n]
        idx = jnp.arange(n, dtype=_I32)
        hi_s, lo_s, idx_s = lax.sort((hi, lo, idx), num_keys=2)
        neq = (hi_s[1:] != hi_s[:-1]) | (lo_s[1:] != lo_s[:-1])
        flags = jnp.concatenate(
            [jnp.zeros((1,), _I32), neq.astype(_I32)])
        ranks = jnp.cumsum(flags, dtype=_I32)
        colors = jnp.zeros((n,), _I32).at[idx_s].set(ranks)

    return colors.astype(jnp.int64)


# speedup vs baseline: 654.1731x; 1.0652x over previous
import functools

import jax
import jax.numpy as jnp
import numpy as np
from jax import lax
from jax.experimental import pallas as pl
from jax.experimental.pallas import tpu as pltpu
from jax.experimental.pallas import tpu_sc as plsc

_NUM_IT = 3

_C_ADD_LO = np.uint32(0x7F4A7C15)
_C_ADD_HI = np.uint32(0x9E3779B9)
_M1_LO = np.uint32(0x1CE4E5B9)
_M1_HI = np.uint32(0xBF58476D)
_M2_LO = np.uint32(0x133111EB)
_M2_HI = np.uint32(0x94D049BB)
_C_NBR = np.uint32(0x1B873593)
_OWN_LO_MUL = np.uint32(0x1B3)

_U32 = jnp.uint32
_I32 = jnp.int32


def _mulhi_u32(a, b):
    m16 = np.uint32(0xFFFF)
    a0 = a & m16
    a1 = a >> np.uint32(16)
    b0 = b & m16
    b1 = b >> np.uint32(16)
    t = a0 * b0
    mid1 = a1 * b0
    mid2 = a0 * b1
    cross = (t >> np.uint32(16)) + (mid1 & m16) + (mid2 & m16)
    return a1 * b1 + (mid1 >> np.uint32(16)) + (mid2 >> np.uint32(16)) + (
        cross >> np.uint32(16))


def _add64(alo, ahi, blo, bhi):
    lo = alo + blo
    carry = (lo < alo).astype(_U32)
    return lo, ahi + bhi + carry


def _mul64_const(alo, ahi, clo, chi):
    lo = alo * clo
    hi = _mulhi_u32(alo, clo) + alo * chi + ahi * clo
    return lo, hi


def _xorshr64(lo, hi, k):
    ku = np.uint32(k)
    kc = np.uint32(32 - k)
    nlo = lo ^ ((lo >> ku) | (hi << kc))
    nhi = hi ^ (hi >> ku)
    return nlo, nhi


def _mix64(lo, hi):
    lo, hi = _add64(lo, hi, _C_ADD_LO, _C_ADD_HI)
    lo, hi = _xorshr64(lo, hi, 30)
    lo, hi = _mul64_const(lo, hi, _M1_LO, _M1_HI)
    lo, hi = _xorshr64(lo, hi, 27)
    lo, hi = _mul64_const(lo, hi, _M2_LO, _M2_HI)
    lo, hi = _xorshr64(lo, hi, 31)
    return lo, hi


def _limbs_body(colors_ref, l0_ref, l1_ref, l2_ref, l3_ref):
    c = colors_ref[...].astype(_U32)
    lo, hi = _mix64(c + _C_NBR, jnp.zeros_like(c))
    m11 = np.uint32(0x7FF)
    l0_ref[...] = lo & m11
    l1_ref[...] = (lo >> np.uint32(11)) & m11
    l2_ref[...] = lo >> np.uint32(22)
    l3_ref[...] = hi


def _sig_body(colors_ref, *refs):
    planes = refs[:16]
    siglo_ref, sighi_ref = refs[16], refs[17]
    agg_lo = None
    agg_hi = None
    for c in range(4):
        s0 = planes[4 * c + 0][...]
        s1 = planes[4 * c + 1][...]
        s2 = planes[4 * c + 2][...]
        s3 = planes[4 * c + 3][...]
        a = s1 << np.uint32(11)
        b = s2 << np.uint32(22)
        lo1 = s0 + a
        c1 = (lo1 < s0).astype(_U32)
        lo2 = lo1 + b
        c2 = (lo2 < lo1).astype(_U32)
        hic = (s1 >> np.uint32(21)) + (s2 >> np.uint32(10)) + c1 + c2 + s3
        if agg_lo is None:
            agg_lo, agg_hi = lo2, hic
        else:
            agg_lo, agg_hi = _add64(agg_lo, agg_hi, lo2, hic)
    col = colors_ref[...].astype(_U32)
    own_lo = col * _OWN_LO_MUL
    own_hi = col << np.uint32(8)
    vlo, vhi = _add64(own_lo, own_hi, agg_lo, agg_hi)
    slo, shi = _mix64(vlo, vhi)
    siglo_ref[...] = slo
    sighi_ref[...] = shi


def _tc_limbs(colors2d):
    shp = jax.ShapeDtypeStruct(colors2d.shape, _U32)
    return pl.pallas_call(
        _limbs_body,
        out_shape=(shp, shp, shp, shp),
    )(colors2d)


def _tc_sig(colors2d, planes):
    shp = jax.ShapeDtypeStruct(colors2d.shape, _U32)
    return pl.pallas_call(
        _sig_body,
        out_shape=(shp, shp),
    )(colors2d, *planes)


@functools.lru_cache(maxsize=None)
def _make_sc_scatter(n_pad, rows_per_tile, e_rows):
    blocks = rows_per_tile // 32
    mesh = plsc.VectorSubcoreMesh(core_axis_name="c", subcore_axis_name="s")
    oshape = jax.ShapeDtypeStruct((4 * n_pad,), _I32)

    @functools.partial(
        pl.kernel,
        mesh=mesh,
        out_type=(oshape, oshape, oshape, oshape),
        scratch_types=[
            pltpu.VMEM((32, 128), _I32),
            pltpu.VMEM((32, 128), _I32),
            [pltpu.VMEM((4096,), _I32)] * 4,
            [pltpu.VMEM_SHARED((2 * n_pad,), _I32)] * 4,
            [pltpu.VMEM_SHARED((n_pad,), _I32)] * 4,
            pltpu.SemaphoreType.DMA,
            pltpu.SemaphoreType.DMA,
        ],
    )
    def sc(t0, t1, t2, t3, rowi, coli, zeros, o0, o1, o2, o3,
           rowbuf, colbuf, gbufs, accs, tss, sem_g, sem_s):
        cid = lax.axis_index("c")
        sid = lax.axis_index("s")
        w = cid * np.int32(16) + sid
        tplanes = [t0, t1, t2, t3]
        outs = [o0, o1, o2, o3]

        zrows = (2 * n_pad) // 16
        z0 = pl.multiple_of(sid * np.int32(zrows), 8)
        trows = n_pad // 16
        tr0 = pl.multiple_of(sid * np.int32(trows), 8)
        for p in range(4):
            pltpu.sync_copy(zeros.at[pl.ds(z0, zrows)],
                            accs[p].at[pl.ds(z0, zrows)])
            pltpu.sync_copy(tplanes[p].at[pl.ds(tr0, trows)],
                            tss[p].at[pl.ds(tr0, trows)])

        plsc.subcore_barrier()
        base = w * np.int32(rows_per_tile)

        def fire_gathers(jj):
            g0 = pl.multiple_of(jj * np.int32(128), 8)
            for p in range(4):
                pltpu.async_copy(tss[p].at[colbuf.at[jj]],
                                 gbufs[p].at[pl.ds(g0, 128)], sem_g)

        def blk(i, r0):
            del i
            r0 = pl.multiple_of(r0, 16)
            pltpu.sync_copy(rowi.at[pl.ds(r0, 32)], rowbuf)
            pltpu.sync_copy(coli.at[pl.ds(r0, 32)], colbuf)
            fire_gathers(np.int32(0))

            def jblk(_, jj):
                g0 = pl.multiple_of(jj * np.int32(128), 8)

                @pl.when(jj < np.int32(31))
                def _():
                    fire_gathers(jj + np.int32(1))

                for p in range(4):
                    pltpu.make_async_copy(
                        tss[p].at[colbuf.at[jj]],
                        gbufs[p].at[pl.ds(g0, 128)], sem_g).wait()
                for p in range(4):
                    pltpu.async_copy(gbufs[p].at[pl.ds(g0, 128)],
                                     accs[p].at[rowbuf.at[jj]], sem_s,
                                     add=True)
                return jj + np.int32(1)

            lax.fori_loop(0, 32, jblk, np.int32(0))
            for p in range(4):
                pltpu.make_async_copy(zeros.at[pl.ds(0, 4096)], gbufs[p],
                                      sem_s).wait()
            return r0 + np.int32(32)

        lax.fori_loop(0, blocks, blk, base)
        plsc.subcore_barrier()
        obase = pl.multiple_of(cid * np.int32(2 * n_pad) + z0, 8)
        for p in range(4):
            pltpu.sync_copy(accs[p].at[pl.ds(z0, zrows)],
                            outs[p].at[pl.ds(obase, zrows)])

    return sc


def _sc_scatter(tplanes, rowi, coli, zeros, n_pad, rows_per_tile):
    fn = _make_sc_scatter(n_pad, rows_per_tile, rowi.shape[0])
    return fn(*tplanes, rowi, coli, zeros)


def kernel(x, edge_index):
    if x.ndim > 1:
        x = jnp.argmax(x, axis=-1)
    n = x.shape[0]
    e = edge_index.shape[1]

    n_pad = ((n + 1023) // 1024) * 1024
    nrows = n_pad // 128
    spare = n_pad - n

    nw = 32
    rows_per_tile = ((e + nw * 2048 - 1) // (nw * 2048)) * 16
    e_rows = nw * rows_per_tile
    e_pad = e_rows * 128
    per_tile = rows_per_tile * 128

    colors = x.astype(_I32)
    row = edge_index[0].astype(_I32)
    col = edge_index[1].astype(_I32)

    npad_e = e_pad - e
    spread = (jnp.arange(npad_e, dtype=_I32) % np.int32(max(spare, 1))
              ) + np.int32(n)
    row_f = jnp.concatenate([row, spread])
    col_f = jnp.concatenate([col, spread])
    epos = jnp.arange(e_pad, dtype=_I32)
    cp = ((epos // np.int32(per_tile)) % np.int32(16)) // np.int32(8)
    row_adj = (row_f + cp * np.int32(n_pad)).reshape(e_rows, 128)
    col_r = col_f.reshape(e_rows, 128)
    zeros = jnp.zeros((2 * n_pad,), _I32)

    colors_pad = jnp.zeros((n_pad,), _I32)

    for _ in range(_NUM_IT):
        colors2d = colors_pad.at[:n].set(colors).reshape(nrows, 128)
        limbs = _tc_limbs(colors2d)
        tplanes = [
            lax.bitcast_convert_type(p, _I32).reshape(-1) for p in limbs
        ]
        out_sc = _sc_scatter(tplanes, row_adj, col_r, zeros, n_pad,
                             rows_per_tile)
        outs_u = [lax.bitcast_convert_type(o, _U32) for o in out_sc]
        planes = []
        for core in range(2):
            for half in range(2):
                off = core * 2 * n_pad + half * n_pad
                for comp in range(4):
                    planes.append(
                        outs_u[comp][off:off + n_pad].reshape(nrows, 128))
        siglo, sighi = _tc_sig(colors2d, planes)
        hi = sighi.reshape(-1)[:n]
        lo = siglo.reshape(-1)[:n]
        idx = jnp.arange(n, dtype=_I32)
        hi_s, lo_s, idx_s = lax.sort((hi, lo, idx), num_keys=2)
        neq = (hi_s[1:] != hi_s[:-1]) | (lo_s[1:] != lo_s[:-1])
        flags = jnp.concatenate(
            [jnp.zeros((1,), _I32), neq.astype(_I32)])
        ranks = jnp.cumsum(flags, dtype=_I32)
        colors = jnp.zeros((n,), _I32).at[idx_s].set(ranks)

    return colors.astype(jnp.int64)


# speedup vs baseline: 832.9438x; 1.2733x over previous
import functools

import jax
import jax.numpy as jnp
import numpy as np
from jax import lax
from jax.experimental import pallas as pl
from jax.experimental.pallas import tpu as pltpu
from jax.experimental.pallas import tpu_sc as plsc

_NUM_IT = 3

_C_ADD_LO = np.uint32(0x7F4A7C15)
_C_ADD_HI = np.uint32(0x9E3779B9)
_M1_LO = np.uint32(0x1CE4E5B9)
_M1_HI = np.uint32(0xBF58476D)
_M2_LO = np.uint32(0x133111EB)
_M2_HI = np.uint32(0x94D049BB)
_C_NBR = np.uint32(0x1B873593)
_OWN_LO_MUL = np.uint32(0x1B3)

_U32 = jnp.uint32
_I32 = jnp.int32


def _mulhi_u32(a, b):
    m16 = np.uint32(0xFFFF)
    a0 = a & m16
    a1 = a >> np.uint32(16)
    b0 = b & m16
    b1 = b >> np.uint32(16)
    t = a0 * b0
    mid1 = a1 * b0
    mid2 = a0 * b1
    cross = (t >> np.uint32(16)) + (mid1 & m16) + (mid2 & m16)
    return a1 * b1 + (mid1 >> np.uint32(16)) + (mid2 >> np.uint32(16)) + (
        cross >> np.uint32(16))


def _add64(alo, ahi, blo, bhi):
    lo = alo + blo
    carry = (lo < alo).astype(_U32)
    return lo, ahi + bhi + carry


def _mul64_const(alo, ahi, clo, chi):
    lo = alo * clo
    hi = _mulhi_u32(alo, clo) + alo * chi + ahi * clo
    return lo, hi


def _xorshr64(lo, hi, k):
    ku = np.uint32(k)
    kc = np.uint32(32 - k)
    nlo = lo ^ ((lo >> ku) | (hi << kc))
    nhi = hi ^ (hi >> ku)
    return nlo, nhi


def _mix64(lo, hi):
    lo, hi = _add64(lo, hi, _C_ADD_LO, _C_ADD_HI)
    lo, hi = _xorshr64(lo, hi, 30)
    lo, hi = _mul64_const(lo, hi, _M1_LO, _M1_HI)
    lo, hi = _xorshr64(lo, hi, 27)
    lo, hi = _mul64_const(lo, hi, _M2_LO, _M2_HI)
    lo, hi = _xorshr64(lo, hi, 31)
    return lo, hi


def _limbs_body(colors_ref, l0_ref, l1_ref, l2_ref, l3_ref):
    c = colors_ref[...].astype(_U32)
    lo, hi = _mix64(c + _C_NBR, jnp.zeros_like(c))
    m11 = np.uint32(0x7FF)
    l0_ref[...] = lo & m11
    l1_ref[...] = (lo >> np.uint32(11)) & m11
    l2_ref[...] = lo >> np.uint32(22)
    l3_ref[...] = hi


def _sig_body(colors_ref, *refs):
    planes = refs[:16]
    siglo_ref, sighi_ref = refs[16], refs[17]
    agg_lo = None
    agg_hi = None
    for c in range(4):
        s0 = planes[4 * c + 0][...]
        s1 = planes[4 * c + 1][...]
        s2 = planes[4 * c + 2][...]
        s3 = planes[4 * c + 3][...]
        a = s1 << np.uint32(11)
        b = s2 << np.uint32(22)
        lo1 = s0 + a
        c1 = (lo1 < s0).astype(_U32)
        lo2 = lo1 + b
        c2 = (lo2 < lo1).astype(_U32)
        hic = (s1 >> np.uint32(21)) + (s2 >> np.uint32(10)) + c1 + c2 + s3
        if agg_lo is None:
            agg_lo, agg_hi = lo2, hic
        else:
            agg_lo, agg_hi = _add64(agg_lo, agg_hi, lo2, hic)
    col = colors_ref[...].astype(_U32)
    own_lo = col * _OWN_LO_MUL
    own_hi = col << np.uint32(8)
    vlo, vhi = _add64(own_lo, own_hi, agg_lo, agg_hi)
    slo, shi = _mix64(vlo, vhi)
    siglo_ref[...] = slo
    sighi_ref[...] = shi


def _tc_limbs(colors2d):
    shp = jax.ShapeDtypeStruct(colors2d.shape, _U32)
    return pl.pallas_call(
        _limbs_body,
        out_shape=(shp, shp, shp, shp),
    )(colors2d)


def _tc_sig(colors2d, planes):
    shp = jax.ShapeDtypeStruct(colors2d.shape, _U32)
    return pl.pallas_call(
        _sig_body,
        out_shape=(shp, shp),
    )(colors2d, *planes)


@functools.lru_cache(maxsize=None)
def _make_sc_scatter(n_pad, rows_per_tile, e_rows):
    blocks = rows_per_tile // 32
    mesh = plsc.VectorSubcoreMesh(core_axis_name="c", subcore_axis_name="s")
    oshape = jax.ShapeDtypeStruct((4 * n_pad,), _I32)

    @functools.partial(
        pl.kernel,
        mesh=mesh,
        out_type=(oshape, oshape, oshape, oshape),
        scratch_types=[
            pltpu.VMEM((32, 128), _I32),
            pltpu.VMEM((32, 128), _I32),
            [pltpu.VMEM((4096,), _I32)] * 4,
            [pltpu.VMEM_SHARED((2 * n_pad,), _I32)] * 4,
            [pltpu.VMEM_SHARED((n_pad,), _I32)] * 4,
            pltpu.SemaphoreType.DMA,
            pltpu.SemaphoreType.DMA,
        ],
    )
    def sc(t0, t1, t2, t3, rowi, coli, zeros, o0, o1, o2, o3,
           rowbuf, colbuf, gbufs, accs, tss, sem_g, sem_s):
        cid = lax.axis_index("c")
        sid = lax.axis_index("s")
        w = cid * np.int32(16) + sid
        tplanes = [t0, t1, t2, t3]
        outs = [o0, o1, o2, o3]

        zrows = (2 * n_pad) // 16
        z0 = pl.multiple_of(sid * np.int32(zrows), 8)
        trows = n_pad // 16
        tr0 = pl.multiple_of(sid * np.int32(trows), 8)
        for p in range(4):
            pltpu.sync_copy(zeros.at[pl.ds(z0, zrows)],
                            accs[p].at[pl.ds(z0, zrows)])
            pltpu.sync_copy(tplanes[p].at[pl.ds(tr0, trows)],
                            tss[p].at[pl.ds(tr0, trows)])

        plsc.subcore_barrier()
        base = w * np.int32(rows_per_tile)

        def fire_gathers(jj):
            g0 = pl.multiple_of(jj * np.int32(128), 8)
            for p in range(4):
                pltpu.async_copy(tss[p].at[colbuf.at[jj]],
                                 gbufs[p].at[pl.ds(g0, 128)], sem_g)

        def blk(i, r0):
            del i
            r0 = pl.multiple_of(r0, 16)
            pltpu.sync_copy(rowi.at[pl.ds(r0, 32)], rowbuf)
            pltpu.sync_copy(coli.at[pl.ds(r0, 32)], colbuf)
            fire_gathers(np.int32(0))

            def jblk(_, jj):
                g0 = pl.multiple_of(jj * np.int32(128), 8)

                @pl.when(jj < np.int32(31))
                def _():
                    fire_gathers(jj + np.int32(1))

                for p in range(4):
                    pltpu.make_async_copy(
                        tss[p].at[colbuf.at[jj]],
                        gbufs[p].at[pl.ds(g0, 128)], sem_g).wait()
                for p in range(4):
                    pltpu.async_copy(gbufs[p].at[pl.ds(g0, 128)],
                                     accs[p].at[rowbuf.at[jj]], sem_s,
                                     add=True)
                return jj + np.int32(1)

            lax.fori_loop(0, 32, jblk, np.int32(0))
            for p in range(4):
                pltpu.make_async_copy(zeros.at[pl.ds(0, 4096)], gbufs[p],
                                      sem_s).wait()
            return r0 + np.int32(32)

        lax.fori_loop(0, blocks, blk, base)
        plsc.subcore_barrier()
        obase = pl.multiple_of(cid * np.int32(2 * n_pad) + z0, 8)
        for p in range(4):
            pltpu.sync_copy(accs[p].at[pl.ds(z0, zrows)],
                            outs[p].at[pl.ds(obase, zrows)])

    return sc


def _sc_scatter(tplanes, rowi, coli, zeros, n_pad, rows_per_tile):
    fn = _make_sc_scatter(n_pad, rows_per_tile, rowi.shape[0])
    return fn(*tplanes, rowi, coli, zeros)


def kernel(x, edge_index):
    if x.ndim > 1:
        x = jnp.argmax(x, axis=-1)
    n = x.shape[0]
    e = edge_index.shape[1]

    n_pad = ((n + 1023) // 1024) * 1024
    nrows = n_pad // 128
    spare = n_pad - n

    nw = 32
    rows_per_tile = ((e + nw * 2048 - 1) // (nw * 2048)) * 16
    e_rows = nw * rows_per_tile
    e_pad = e_rows * 128
    per_tile = rows_per_tile * 128

    colors = x.astype(_I32)
    row = edge_index[0].astype(_I32)
    col = edge_index[1].astype(_I32)

    npad_e = e_pad - e
    spread = (jnp.arange(npad_e, dtype=_I32) % np.int32(max(spare, 1))
              ) + np.int32(n)
    row_f = jnp.concatenate([row, spread])
    col_f = jnp.concatenate([col, spread])
    epos = jnp.arange(e_pad, dtype=_I32)
    cp = ((epos // np.int32(per_tile)) % np.int32(16)) // np.int32(8)
    row_adj = (row_f + cp * np.int32(n_pad)).reshape(e_rows, 128)
    col_r = col_f.reshape(e_rows, 128)
    zeros = jnp.zeros((2 * n_pad,), _I32)

    colors_pad = jnp.zeros((n_pad,), _I32)

    for _ in range(_NUM_IT):
        colors2d = colors_pad.at[:n].set(colors).reshape(nrows, 128)
        limbs = _tc_limbs(colors2d)
        tplanes = [
            lax.bitcast_convert_type(p, _I32).reshape(-1) for p in limbs
        ]
        out_sc = _sc_scatter(tplanes, row_adj, col_r, zeros, n_pad,
                             rows_per_tile)
        outs_u = [lax.bitcast_convert_type(o, _U32) for o in out_sc]
        planes = []
        for core in range(2):
            for half in range(2):
                off = core * 2 * n_pad + half * n_pad
                for comp in range(4):
                    planes.append(
                        outs_u[comp][off:off + n_pad].reshape(nrows, 128))
        siglo, sighi = _tc_sig(colors2d, planes)
        hi = sighi.reshape(-1)[:n]
        lo = siglo.reshape(-1)[:n]
        idx = jnp.arange(n, dtype=_I32)
        hi_s, lo_s, idx_s = lax.sort((hi, lo, idx), num_keys=2)
        neq = (hi_s[1:] != hi_s[:-1]) | (lo_s[1:] != lo_s[:-1])
        flags = jnp.concatenate(
            [jnp.zeros((1,), _I32), neq.astype(_I32)])
        ranks = jnp.cumsum(flags, dtype=_I32)
        _, colors = lax.sort((idx_s, ranks), num_keys=1)

    return colors.astype(jnp.int64)


# speedup vs baseline: 892.4206x; 1.0714x over previous
import functools

import jax
import jax.numpy as jnp
import numpy as np
from jax import lax
from jax.experimental import pallas as pl
from jax.experimental.pallas import tpu as pltpu
from jax.experimental.pallas import tpu_sc as plsc

_NUM_IT = 3

_C_ADD_LO = np.uint32(0x7F4A7C15)
_C_ADD_HI = np.uint32(0x9E3779B9)
_M1_LO = np.uint32(0x1CE4E5B9)
_M1_HI = np.uint32(0xBF58476D)
_M2_LO = np.uint32(0x133111EB)
_M2_HI = np.uint32(0x94D049BB)
_C_NBR = np.uint32(0x1B873593)
_OWN_LO_MUL = np.uint32(0x1B3)

_U32 = jnp.uint32
_I32 = jnp.int32


def _mulhi_u32(a, b):
    m16 = np.uint32(0xFFFF)
    a0 = a & m16
    a1 = a >> np.uint32(16)
    b0 = b & m16
    b1 = b >> np.uint32(16)
    t = a0 * b0
    mid1 = a1 * b0
    mid2 = a0 * b1
    cross = (t >> np.uint32(16)) + (mid1 & m16) + (mid2 & m16)
    return a1 * b1 + (mid1 >> np.uint32(16)) + (mid2 >> np.uint32(16)) + (
        cross >> np.uint32(16))


def _add64(alo, ahi, blo, bhi):
    lo = alo + blo
    carry = (lo < alo).astype(_U32)
    return lo, ahi + bhi + carry


def _mul64_const(alo, ahi, clo, chi):
    lo = alo * clo
    hi = _mulhi_u32(alo, clo) + alo * chi + ahi * clo
    return lo, hi


def _xorshr64(lo, hi, k):
    ku = np.uint32(k)
    kc = np.uint32(32 - k)
    nlo = lo ^ ((lo >> ku) | (hi << kc))
    nhi = hi ^ (hi >> ku)
    return nlo, nhi


def _mix64(lo, hi):
    lo, hi = _add64(lo, hi, _C_ADD_LO, _C_ADD_HI)
    lo, hi = _xorshr64(lo, hi, 30)
    lo, hi = _mul64_const(lo, hi, _M1_LO, _M1_HI)
    lo, hi = _xorshr64(lo, hi, 27)
    lo, hi = _mul64_const(lo, hi, _M2_LO, _M2_HI)
    lo, hi = _xorshr64(lo, hi, 31)
    return lo, hi


def _limbs_body(colors_ref, l0_ref, l1_ref, l2_ref, l3_ref):
    c = colors_ref[...].astype(_U32)
    lo, hi = _mix64(c + _C_NBR, jnp.zeros_like(c))
    m11 = np.uint32(0x7FF)
    l0_ref[...] = lo & m11
    l1_ref[...] = (lo >> np.uint32(11)) & m11
    l2_ref[...] = lo >> np.uint32(22)
    l3_ref[...] = hi


def _sig_body(colors_ref, *refs):
    planes = refs[:16]
    siglo_ref, sighi_ref = refs[16], refs[17]
    agg_lo = None
    agg_hi = None
    for c in range(4):
        s0 = planes[4 * c + 0][...]
        s1 = planes[4 * c + 1][...]
        s2 = planes[4 * c + 2][...]
        s3 = planes[4 * c + 3][...]
        a = s1 << np.uint32(11)
        b = s2 << np.uint32(22)
        lo1 = s0 + a
        c1 = (lo1 < s0).astype(_U32)
        lo2 = lo1 + b
        c2 = (lo2 < lo1).astype(_U32)
        hic = (s1 >> np.uint32(21)) + (s2 >> np.uint32(10)) + c1 + c2 + s3
        if agg_lo is None:
            agg_lo, agg_hi = lo2, hic
        else:
            agg_lo, agg_hi = _add64(agg_lo, agg_hi, lo2, hic)
    col = colors_ref[...].astype(_U32)
    own_lo = col * _OWN_LO_MUL
    own_hi = col << np.uint32(8)
    vlo, vhi = _add64(own_lo, own_hi, agg_lo, agg_hi)
    slo, shi = _mix64(vlo, vhi)
    siglo_ref[...] = slo
    sighi_ref[...] = shi


def _tc_limbs(colors2d):
    shp = jax.ShapeDtypeStruct(colors2d.shape, _U32)
    return pl.pallas_call(
        _limbs_body,
        out_shape=(shp, shp, shp, shp),
    )(colors2d)


def _tc_sig(colors2d, planes):
    shp = jax.ShapeDtypeStruct(colors2d.shape, _U32)
    return pl.pallas_call(
        _sig_body,
        out_shape=(shp, shp),
    )(colors2d, *planes)


@functools.lru_cache(maxsize=None)
def _make_sc_scatter(n_pad, rows_per_tile, e_rows):
    blocks = rows_per_tile // 32
    mesh = plsc.VectorSubcoreMesh(core_axis_name="c", subcore_axis_name="s")
    oshape = jax.ShapeDtypeStruct((4 * n_pad,), _I32)

    @functools.partial(
        pl.kernel,
        mesh=mesh,
        out_type=(oshape, oshape, oshape, oshape),
        scratch_types=[
            pltpu.VMEM((64, 128), _I32),
            pltpu.VMEM((64, 128), _I32),
            [pltpu.VMEM((4096,), _I32)] * 4,
            [pltpu.VMEM_SHARED((2 * n_pad,), _I32)] * 4,
            [pltpu.VMEM_SHARED((n_pad,), _I32)] * 4,
            pltpu.SemaphoreType.DMA,
            pltpu.SemaphoreType.DMA,
            pltpu.SemaphoreType.DMA,
        ],
    )
    def sc(t0, t1, t2, t3, rowi, coli, zeros, o0, o1, o2, o3,
           rowbuf, colbuf, gbufs, accs, tss, sem_g, sem_s, sem_l):
        cid = lax.axis_index("c")
        sid = lax.axis_index("s")
        w = cid * np.int32(16) + sid
        tplanes = [t0, t1, t2, t3]
        outs = [o0, o1, o2, o3]

        zrows = (2 * n_pad) // 16
        z0 = pl.multiple_of(sid * np.int32(zrows), 8)
        trows = n_pad // 16
        tr0 = pl.multiple_of(sid * np.int32(trows), 8)
        for p in range(4):
            pltpu.sync_copy(zeros.at[pl.ds(z0, zrows)],
                            accs[p].at[pl.ds(z0, zrows)])
            pltpu.sync_copy(tplanes[p].at[pl.ds(tr0, trows)],
                            tss[p].at[pl.ds(tr0, trows)])

        plsc.subcore_barrier()
        base = w * np.int32(rows_per_tile)
        limit = base + np.int32(rows_per_tile - 32)

        def stage(r0, h):
            pltpu.async_copy(rowi.at[pl.ds(r0, 32)],
                             rowbuf.at[pl.ds(h, 32)], sem_l)
            pltpu.async_copy(coli.at[pl.ds(r0, 32)],
                             colbuf.at[pl.ds(h, 32)], sem_l)

        def stage_wait(r0, h):
            pltpu.make_async_copy(rowi.at[pl.ds(r0, 32)],
                                  rowbuf.at[pl.ds(h, 32)], sem_l).wait()
            pltpu.make_async_copy(coli.at[pl.ds(r0, 32)],
                                  colbuf.at[pl.ds(h, 32)], sem_l).wait()

        def fire_gathers(h, jj):
            g0 = pl.multiple_of(jj * np.int32(128), 8)
            for p in range(4):
                pltpu.async_copy(tss[p].at[colbuf.at[h + jj]],
                                 gbufs[p].at[pl.ds(g0, 128)], sem_g)

        stage(base, np.int32(0))

        def blk(i, carry):
            del i
            r0, h = carry
            r0 = pl.multiple_of(r0, 16)
            stage_wait(r0, h)

            @pl.when(r0 < limit)
            def _():
                stage(r0 + np.int32(32), np.int32(32) - h)

            fire_gathers(h, np.int32(0))
            fire_gathers(h, np.int32(1))

            def jblk(_, jj):
                g0 = pl.multiple_of(jj * np.int32(128), 8)

                @pl.when(jj < np.int32(30))
                def _():
                    fire_gathers(h, jj + np.int32(2))

                for p in range(4):
                    pltpu.make_async_copy(
                        tss[p].at[colbuf.at[h + jj]],
                        gbufs[p].at[pl.ds(g0, 128)], sem_g).wait()
                for p in range(4):
                    pltpu.async_copy(gbufs[p].at[pl.ds(g0, 128)],
                                     accs[p].at[rowbuf.at[h + jj]], sem_s,
                                     add=True)
                return jj + np.int32(1)

            lax.fori_loop(0, 32, jblk, np.int32(0))
            for p in range(4):
                pltpu.make_async_copy(zeros.at[pl.ds(0, 4096)], gbufs[p],
                                      sem_s).wait()
            return (r0 + np.int32(32), np.int32(32) - h)

        lax.fori_loop(0, blocks, blk, (base, np.int32(0)))
        plsc.subcore_barrier()
        obase = pl.multiple_of(cid * np.int32(2 * n_pad) + z0, 8)
        for p in range(4):
            pltpu.sync_copy(accs[p].at[pl.ds(z0, zrows)],
                            outs[p].at[pl.ds(obase, zrows)])

    return sc


def _sc_scatter(tplanes, rowi, coli, zeros, n_pad, rows_per_tile):
    fn = _make_sc_scatter(n_pad, rows_per_tile, rowi.shape[0])
    return fn(*tplanes, rowi, coli, zeros)


def kernel(x, edge_index):
    if x.ndim > 1:
        x = jnp.argmax(x, axis=-1)
    n = x.shape[0]
    e = edge_index.shape[1]

    n_pad = ((n + 1023) // 1024) * 1024
    nrows = n_pad // 128
    spare = n_pad - n

    nw = 32
    rows_per_tile = ((e + nw * 2048 - 1) // (nw * 2048)) * 16
    e_rows = nw * rows_per_tile
    e_pad = e_rows * 128
    per_tile = rows_per_tile * 128

    colors = x.astype(_I32)
    row = edge_index[0].astype(_I32)
    col = edge_index[1].astype(_I32)

    npad_e = e_pad - e
    spread = (jnp.arange(npad_e, dtype=_I32) % np.int32(max(spare, 1))
              ) + np.int32(n)
    row_f = jnp.concatenate([row, spread])
    col_f = jnp.concatenate([col, spread])
    erow = jnp.arange(e_rows, dtype=_I32)
    cp = ((erow // np.int32(rows_per_tile)) % np.int32(16)) // np.int32(8)
    row_adj = row_f.reshape(e_rows, 128) + (cp * np.int32(n_pad))[:, None]
    col_r = col_f.reshape(e_rows, 128)
    zeros = jnp.zeros((2 * n_pad,), _I32)

    colors_pad = jnp.zeros((n_pad,), _I32)

    for _ in range(_NUM_IT):
        colors2d = colors_pad.at[:n].set(colors).reshape(nrows, 128)
        limbs = _tc_limbs(colors2d)
        tplanes = [
            lax.bitcast_convert_type(p, _I32).reshape(-1) for p in limbs
        ]
        out_sc = _sc_scatter(tplanes, row_adj, col_r, zeros, n_pad,
                             rows_per_tile)
        outs_u = [lax.bitcast_convert_type(o, _U32) for o in out_sc]
        planes = []
        for core in range(2):
            for half in range(2):
                off = core * 2 * n_pad + half * n_pad
                for comp in range(4):
                    planes.append(
                        outs_u[comp][off:off + n_pad].reshape(nrows, 128))
        siglo, sighi = _tc_sig(colors2d, planes)
        hi = sighi.reshape(-1)[:n]
        lo = siglo.reshape(-1)[:n]
        idx = jnp.arange(n, dtype=_I32)
        hi_s, lo_s, idx_s = lax.sort((hi, lo, idx), num_keys=2)
        neq = (hi_s[1:] != hi_s[:-1]) | (lo_s[1:] != lo_s[:-1])
        flags = jnp.concatenate(
            [jnp.zeros((1,), _I32), neq.astype(_I32)])
        ranks = jnp.cumsum(flags, dtype=_I32)
        _, colors = lax.sort((idx_s, ranks), num_keys=1)

    return colors.astype(jnp.int64)
